# baseline (speedup 1.0000x reference)
import functools

import jax
import jax.numpy as jnp
from jax import lax
from jax.experimental import pallas as pl
from jax.experimental.pallas import tpu as pltpu
from jax.experimental.pallas import tpu_sc as plsc

D_MODEL = 1024
D_VOCAB = 100000
BATCH = 4
SEQ = 8192
B_TOTAL = BATCH * SEQ

_BV = 512


def _transpose_body(in_ref, out_ref):
    out_ref[...] = in_ref[...].T


def _transpose_table(w):
    nblk = pl.cdiv(D_VOCAB, _BV)
    return pl.pallas_call(
        _transpose_body,
        grid=(nblk,),
        in_specs=[pl.BlockSpec((D_MODEL, _BV), lambda i: (0, i))],
        out_specs=pl.BlockSpec((_BV, D_MODEL), lambda i: (i, 0)),
        out_shape=jax.ShapeDtypeStruct((D_VOCAB, D_MODEL), jnp.float32),
    )(w)


_NC = 2
_NS = 16
_NW = _NC * _NS
_B_PER_W = B_TOTAL // _NW
_CH = 64
_NCHUNK = _B_PER_W // _CH


def _sc_gather(table, idx):
    mesh = plsc.VectorSubcoreMesh(core_axis_name="c", subcore_axis_name="s")

    @functools.partial(
        pl.kernel,
        mesh=mesh,
        out_type=jax.ShapeDtypeStruct((B_TOTAL, D_MODEL), jnp.float32),
        scratch_types=[
            pltpu.VMEM((_CH,), jnp.int32),
            pltpu.VMEM((_CH, D_MODEL), jnp.float32),
            pltpu.SemaphoreType.DMA,
        ],
    )
    def k(table_hbm, idx_hbm, out_hbm, idx_v, rows_v, sem):
        wid = lax.axis_index("s") * _NC + lax.axis_index("c")
        base = wid * _B_PER_W

        def body(i, carry):
            off = base + i * _CH
            pltpu.sync_copy(idx_hbm.at[pl.ds(off, _CH)], idx_v)
            pltpu.async_copy(table_hbm.at[idx_v], rows_v, sem).wait()
            pltpu.sync_copy(rows_v, out_hbm.at[pl.ds(off, _CH)])
            return carry

        lax.fori_loop(0, _NCHUNK, body, 0)

    return k(table, idx)


def kernel(tokens, W_E):
    idx = tokens.reshape(-1).astype(jnp.int32)
    w_t = _transpose_table(W_E)
    out = _sc_gather(w_t, idx)
    return out.reshape(BATCH, SEQ, D_MODEL)
